# Initial kernel scaffold; baseline (speedup 1.0000x reference)
import jax
import jax.numpy as jnp
from jax.experimental import pallas as pl


def kernel(features, dist, adj_relative_cos, W_conv1, b_conv1, W_agg1, b_agg1, W_conv2, b_conv2):
    raise NotImplementedError("write your pallas kernel here")


# speedup vs baseline: 1.5243x; 1.5243x over previous
import jax
import jax.numpy as jnp
from jax.experimental import pallas as pl

N = 4096
D_IN = 256
D_HID = 256
D_OUT = 128
D_CAT = D_HID + D_IN
BI = 512
EPS = 1e-8


def _feats_body(feat_ref, w1_ref, w2_ref, h1_ref, h2_ref):
    f = feat_ref[...]
    h1_ref[...] = jnp.dot(f, w1_ref[...], preferred_element_type=jnp.float32)
    h2_ref[...] = jnp.dot(f, w2_ref[...], preferred_element_type=jnp.float32)


def _agg_body(dist_ref, cos_ref, h1_ref, h2_ref, b1_ref, b2_ref, wo_ref, y_ref):
    d = dist_ref[...]
    c = cos_ref[...]
    inv_sd = 1.0 / (jnp.sum(d, axis=1, keepdims=True) + EPS)
    inv_sc = 1.0 / (jnp.sum(jnp.abs(c), axis=1, keepdims=True) + EPS)
    x1 = jnp.dot(d, h1_ref[...], preferred_element_type=jnp.float32)
    x1 = jnp.maximum(x1 * inv_sd + b1_ref[...], 0.0)
    x2 = jnp.dot(c, h2_ref[...], preferred_element_type=jnp.float32)
    x2 = jnp.maximum(x2 * inv_sc + b2_ref[...], 0.0)
    y_ref[...] = (
        jnp.dot(x1, wo_ref[0:D_HID, :], preferred_element_type=jnp.float32)
        + jnp.dot(x2, wo_ref[D_HID:D_CAT, :], preferred_element_type=jnp.float32)
    )


def _out_body(dist_ref, y_ref, bo_ref, out_ref):
    d = dist_ref[...]
    inv_sd = 1.0 / (jnp.sum(d, axis=1, keepdims=True) + EPS)
    acc = jnp.dot(d, y_ref[...], preferred_element_type=jnp.float32)
    out_ref[...] = acc * inv_sd + bo_ref[...]


def kernel(features, dist, adj_relative_cos, W_conv1, b_conv1, W_agg1, b_agg1, W_conv2, b_conv2):
    h1, h2 = pl.pallas_call(
        _feats_body,
        out_shape=(
            jax.ShapeDtypeStruct((N, D_HID), jnp.float32),
            jax.ShapeDtypeStruct((N, D_IN), jnp.float32),
        ),
    )(features, W_conv1, W_agg1)

    n_blocks = N // BI
    full = lambda i: (0, 0)
    row_blk = lambda i: (i, 0)

    y = pl.pallas_call(
        _agg_body,
        grid=(n_blocks,),
        in_specs=[
            pl.BlockSpec((BI, N), row_blk),
            pl.BlockSpec((BI, N), row_blk),
            pl.BlockSpec((N, D_HID), full),
            pl.BlockSpec((N, D_IN), full),
            pl.BlockSpec((1, D_HID), full),
            pl.BlockSpec((1, D_IN), full),
            pl.BlockSpec((D_CAT, D_OUT), full),
        ],
        out_specs=pl.BlockSpec((BI, D_OUT), row_blk),
        out_shape=jax.ShapeDtypeStruct((N, D_OUT), jnp.float32),
    )(dist, adj_relative_cos, h1, h2,
      b_conv1.reshape(1, D_HID), b_agg1.reshape(1, D_IN), W_conv2)

    out = pl.pallas_call(
        _out_body,
        grid=(n_blocks,),
        in_specs=[
            pl.BlockSpec((BI, N), row_blk),
            pl.BlockSpec((N, D_OUT), full),
            pl.BlockSpec((1, D_OUT), full),
        ],
        out_specs=pl.BlockSpec((BI, D_OUT), row_blk),
        out_shape=jax.ShapeDtypeStruct((N, D_OUT), jnp.float32),
    )(dist, y, b_conv2.reshape(1, D_OUT))

    return out


# speedup vs baseline: 1.7474x; 1.1464x over previous
import jax
import jax.numpy as jnp
from jax.experimental import pallas as pl
from jax.experimental.pallas import tpu as pltpu

N = 4096
D_IN = 256
D_HID = 256
D_OUT = 128
D_CAT = D_HID + D_IN
BI = 512
NB = N // BI
EPS = 1e-8


def _body(dist_ref, cos_ref, feat_ref, w1_ref, wa_ref, b1_ref, b2_ref,
          wo_ref, bo_ref, out_ref, h1_ref, h2_ref, y_ref, inv_sd_ref):
    i = pl.program_id(0)

    @pl.when(i == 0)
    def _feats():
        f = feat_ref[...]
        h1_ref[...] = jnp.dot(f, w1_ref[...], preferred_element_type=jnp.float32)
        h2_ref[...] = jnp.dot(f, wa_ref[...], preferred_element_type=jnp.float32)

    @pl.when((i >= 1) & (i <= NB))
    def _agg():
        j = i - 1
        d = dist_ref[...]
        c = cos_ref[...]
        inv_d = 1.0 / (jnp.sum(d, axis=1, keepdims=True) + EPS)
        inv_c = 1.0 / (jnp.sum(jnp.abs(c), axis=1, keepdims=True) + EPS)
        x1 = jnp.dot(d, h1_ref[...], preferred_element_type=jnp.float32)
        x1 = jnp.maximum(x1 * inv_d + b1_ref[...], 0.0)
        x2 = jnp.dot(c, h2_ref[...], preferred_element_type=jnp.float32)
        x2 = jnp.maximum(x2 * inv_c + b2_ref[...], 0.0)
        y_ref[pl.ds(j * BI, BI), :] = (
            jnp.dot(x1, wo_ref[0:D_HID, :], preferred_element_type=jnp.float32)
            + jnp.dot(x2, wo_ref[D_HID:D_CAT, :], preferred_element_type=jnp.float32)
        )
        inv_sd_ref[pl.ds(j * BI, BI), :] = inv_d

    @pl.when(i > NB)
    def _out():
        j = i - (NB + 1)
        d = dist_ref[...]
        acc = jnp.dot(d, y_ref[...], preferred_element_type=jnp.float32)
        out_ref[...] = acc * inv_sd_ref[pl.ds(j * BI, BI), :] + bo_ref[...]


def kernel(features, dist, adj_relative_cos, W_conv1, b_conv1, W_agg1, b_agg1, W_conv2, b_conv2):
    full = lambda i: (0, 0)
    dist_idx = lambda i: (jnp.maximum(i - 1, 0) % NB, 0)
    cos_idx = lambda i: (jnp.clip(i - 1, 0, NB - 1), 0)
    out_idx = lambda i: (jnp.maximum(i - (NB + 1), 0), 0)

    out = pl.pallas_call(
        _body,
        grid=(2 * NB + 1,),
        in_specs=[
            pl.BlockSpec((BI, N), dist_idx),
            pl.BlockSpec((BI, N), cos_idx),
            pl.BlockSpec((N, D_IN), full),
            pl.BlockSpec((D_IN, D_HID), full),
            pl.BlockSpec((D_IN, D_IN), full),
            pl.BlockSpec((1, D_HID), full),
            pl.BlockSpec((1, D_IN), full),
            pl.BlockSpec((D_CAT, D_OUT), full),
            pl.BlockSpec((1, D_OUT), full),
        ],
        out_specs=pl.BlockSpec((BI, D_OUT), out_idx),
        out_shape=jax.ShapeDtypeStruct((N, D_OUT), jnp.float32),
        scratch_shapes=[
            pltpu.VMEM((N, D_HID), jnp.float32),
            pltpu.VMEM((N, D_IN), jnp.float32),
            pltpu.VMEM((N, D_OUT), jnp.float32),
            pltpu.VMEM((N, 1), jnp.float32),
        ],
    )(dist, adj_relative_cos, features, W_conv1, W_agg1,
      b_conv1.reshape(1, D_HID), b_agg1.reshape(1, D_IN),
      W_conv2, b_conv2.reshape(1, D_OUT))

    return out


# speedup vs baseline: 1.8591x; 1.0639x over previous
import jax
import jax.numpy as jnp
from jax.experimental import pallas as pl
from jax.experimental.pallas import tpu as pltpu

N = 4096
D_IN = 256
D_HID = 256
D_OUT = 128
D_CAT = D_HID + D_IN
BI = 256
NB = N // BI
EPS = 1e-8


def _feats_body(feat_ref, w1_ref, wa_ref, h1_ref, h2_ref):
    f = feat_ref[...]
    h1_ref[...] = jnp.dot(f, w1_ref[...], preferred_element_type=jnp.float32).astype(jnp.bfloat16)
    h2_ref[...] = jnp.dot(f, wa_ref[...], preferred_element_type=jnp.float32).astype(jnp.bfloat16)


def _main_body(dist_ref, cos_ref, h1_ref, h2_ref, b1_ref, b2_ref, wo_ref, bo_ref,
               out_ref, db_ref, y_ref, inv_sd_ref):
    i = pl.program_id(0)

    @pl.when(i < NB)
    def _agg():
        d = dist_ref[...]
        c = cos_ref[...]
        inv_d = 1.0 / (jnp.sum(d, axis=1, keepdims=True) + EPS)
        inv_c = 1.0 / (jnp.sum(jnp.abs(c), axis=1, keepdims=True) + EPS)
        db = d.astype(jnp.bfloat16)
        db_ref[pl.ds(i * BI, BI), :] = db
        x1 = jnp.dot(db, h1_ref[...], preferred_element_type=jnp.float32)
        x1 = jnp.maximum(x1 * inv_d + b1_ref[...], 0.0)
        x2 = jnp.dot(c.astype(jnp.bfloat16), h2_ref[...], preferred_element_type=jnp.float32)
        x2 = jnp.maximum(x2 * inv_c + b2_ref[...], 0.0)
        yblk = (
            jnp.dot(x1, wo_ref[0:D_HID, :], preferred_element_type=jnp.float32)
            + jnp.dot(x2, wo_ref[D_HID:D_CAT, :], preferred_element_type=jnp.float32)
        )
        y_ref[pl.ds(i * BI, BI), :] = yblk.astype(jnp.bfloat16)
        inv_sd_ref[pl.ds(i * BI, BI), :] = inv_d

    @pl.when(i >= NB)
    def _out():
        j = i - NB
        db = db_ref[pl.ds(j * BI, BI), :]
        acc = jnp.dot(db, y_ref[...], preferred_element_type=jnp.float32)
        out_ref[...] = acc * inv_sd_ref[pl.ds(j * BI, BI), :] + bo_ref[...]


def kernel(features, dist, adj_relative_cos, W_conv1, b_conv1, W_agg1, b_agg1, W_conv2, b_conv2):
    full = lambda i: (0, 0)

    h1, h2 = pl.pallas_call(
        _feats_body,
        out_shape=(
            jax.ShapeDtypeStruct((N, D_HID), jnp.bfloat16),
            jax.ShapeDtypeStruct((N, D_IN), jnp.bfloat16),
        ),
    )(features, W_conv1, W_agg1)

    adj_idx = lambda i: (jnp.clip(i, 0, NB - 1), 0)
    out_idx = lambda i: (jnp.maximum(i - NB, 0), 0)

    out = pl.pallas_call(
        _main_body,
        grid=(2 * NB,),
        in_specs=[
            pl.BlockSpec((BI, N), adj_idx),
            pl.BlockSpec((BI, N), adj_idx),
            pl.BlockSpec((N, D_HID), full),
            pl.BlockSpec((N, D_IN), full),
            pl.BlockSpec((1, D_HID), full),
            pl.BlockSpec((1, D_IN), full),
            pl.BlockSpec((D_CAT, D_OUT), full),
            pl.BlockSpec((1, D_OUT), full),
        ],
        out_specs=pl.BlockSpec((BI, D_OUT), out_idx),
        out_shape=jax.ShapeDtypeStruct((N, D_OUT), jnp.float32),
        scratch_shapes=[
            pltpu.VMEM((N, N), jnp.bfloat16),
            pltpu.VMEM((N, D_OUT), jnp.bfloat16),
            pltpu.VMEM((N, 1), jnp.float32),
        ],
    )(dist, adj_relative_cos, h1, h2,
      b_conv1.reshape(1, D_HID), b_agg1.reshape(1, D_IN),
      W_conv2, b_conv2.reshape(1, D_OUT))

    return out
